# Initial kernel scaffold; baseline (speedup 1.0000x reference)
import jax
import jax.numpy as jnp
from jax.experimental import pallas as pl


def kernel(x, W1, g1, b1, W2, g2, b2, W3, g3, b3, W4, g4, b4, W5, g5, b5):
    raise NotImplementedError("write your pallas kernel here")


# speedup vs baseline: 5.1381x; 5.1381x over previous
import functools

import jax
import jax.numpy as jnp
from jax import lax
from jax.experimental import pallas as pl
from jax.experimental.pallas import tpu as pltpu
from jax.experimental.pallas import tpu_sc as plsc

_K = 20
_KP = 32
_N = 1024
_B = 8
_NW = 32
_NC = 2
_CH = 16
_T = 128
_NT = _N // _T
_EPS = 1e-5
_CNT = float(_B * _N * _K)

_DN = (((1,), (0,)), ((), ()))


def _bn_scales(pstat_ref, g_ref, b_ref, cnt):
    s0 = pstat_ref[0, 0:1, :]
    s1 = pstat_ref[0, 1:2, :]
    for w in range(1, _B):
        s0 = s0 + pstat_ref[w, 0:1, :]
        s1 = s1 + pstat_ref[w, 1:2, :]
    m = s0 / cnt
    v = s1 / cnt - m * m
    return m, v


def _normalize(y, m, v, g_ref, b_ref):
    z = (y - m) / jnp.sqrt(v + _EPS) * g_ref[...] + b_ref[...]
    return jnp.where(z > 0, z, 0.2 * z)


def _knn_core(xtp, b, idx_ref, af_ref):
    n = xtp.shape[0]
    gram = lax.dot_general(xtp, xtp, (((1,), (1,)), ((), ())),
                           preferred_element_type=jnp.float32)
    rows = lax.broadcasted_iota(jnp.int32, (n, n), 0)
    cols = lax.broadcasted_iota(jnp.int32, (n, n), 1)
    eyef = (rows == cols).astype(jnp.float32)
    xs = jnp.sum(xtp * xtp, axis=1, keepdims=True)
    xsr = lax.dot_general(xs, eyef, (((0,), (0,)), ((), ())),
                          preferred_element_type=jnp.float32,
                          precision=lax.Precision.HIGHEST)
    inner = -2.0 * gram
    pd = (-xs) - inner - xsr
    af_ref[...] = jnp.zeros((n, _KP), jnp.float32)
    for kk in range(_K):
        mrow = jnp.max(pd, axis=1, keepdims=True)
        j = jnp.min(jnp.where(pd == mrow, cols, n), axis=1, keepdims=True)
        af_ref[:, kk:kk + 1] = j.astype(jnp.float32)
        pd = jnp.where(cols == j, -jnp.inf, pd)
    idxt = lax.dot_general(af_ref[...], eyef, (((0,), (0,)), ((), ())),
                           preferred_element_type=jnp.float32,
                           precision=lax.Precision.HIGHEST)
    idx_ref[0] = idxt.astype(jnp.int32) + b * n


def _a1_body(xtp_ref, idx_ref, af_ref):
    _knn_core(xtp_ref[0], pl.program_id(0), idx_ref, af_ref)


def _a_body(c, mx_ref, pstat_ref, g_ref, b_ref, xt_ref, idx_ref, af_ref):
    b = pl.program_id(0)
    m, v = _bn_scales(pstat_ref, g_ref, b_ref, _CNT)
    xtv = _normalize(mx_ref[0], m, v, g_ref, b_ref)
    xt_ref[0, :, 0:c] = xtv
    if c < 128:
        xt_ref[0, :, c:128] = jnp.zeros((_N, 128 - c), jnp.float32)
    _knn_core(xt_ref[0], b, idx_ref, af_ref)


def _run_a1(xtp):
    return pl.pallas_call(
        _a1_body,
        grid=(_B,),
        in_specs=[pl.BlockSpec((1, _N, 128), lambda b: (b, 0, 0))],
        out_specs=pl.BlockSpec((1, _KP, _N), lambda b: (b, 0, 0)),
        out_shape=jax.ShapeDtypeStruct((_B, _KP, _N), jnp.int32),
        scratch_shapes=[pltpu.VMEM((_N, _KP), jnp.float32)],
    )(xtp)


def _run_a(mx, pstat, g, b):
    c = mx.shape[2]
    return pl.pallas_call(
        functools.partial(_a_body, c),
        grid=(_B,),
        in_specs=[
            pl.BlockSpec((1, _N, c), lambda b: (b, 0, 0)),
            pl.BlockSpec((_B, 8, c), lambda b: (0, 0, 0)),
            pl.BlockSpec((1, c), lambda b: (0, 0)),
            pl.BlockSpec((1, c), lambda b: (0, 0)),
        ],
        out_specs=[
            pl.BlockSpec((1, _N, 128), lambda b: (b, 0, 0)),
            pl.BlockSpec((1, _KP, _N), lambda b: (b, 0, 0)),
        ],
        out_shape=[
            jax.ShapeDtypeStruct((_B, _N, 128), jnp.float32),
            jax.ShapeDtypeStruct((_B, _KP, _N), jnp.int32),
        ],
        scratch_shapes=[pltpu.VMEM((_N, _KP), jnp.float32)],
    )(mx, pstat, g, b)


def _sc_body(cw, xt_hbm, idx_hbm, diff_hbm, idxb, rows, cb, diffb, gsem):
    seg = cw // 16
    ppw = (_B * _N) // _NW
    nch = ppw // _CH
    wid = lax.axis_index("s") * _NC + lax.axis_index("c")
    base = wid * ppw
    bb = base // _N
    nb = base - bb * _N
    pltpu.sync_copy(idx_hbm.at[bb, pl.ds(0, 24), pl.ds(nb, ppw)], idxb)

    def chunk(ch, carry):
        p0 = base + ch * _CH
        pltpu.sync_copy(xt_hbm.at[pl.ds(p0, _CH)], cb)
        handles = [pltpu.async_copy(
            xt_hbm.at[idxb.at[kk, pl.ds(ch * _CH, _CH)]], rows.at[kk], gsem)
            for kk in range(_K)]
        for h in handles:
            h.wait()

        def point(n, c2):
            for sg in range(seg):
                sl = pl.ds(sg * 16, 16)
                cv = cb[n, sl]
                for kk in range(_K):
                    diffb[kk, n, sl] = rows[kk, n, sl] - cv
            return c2

        lax.fori_loop(0, _CH, point, 0)
        pltpu.sync_copy(diffb, diff_hbm.at[pl.ds(0, _K), pl.ds(p0, _CH)])
        return carry

    lax.fori_loop(0, nch, chunk, 0)


@functools.cache
def _make_sc_gather(cw):
    mesh = plsc.VectorSubcoreMesh(core_axis_name="c", subcore_axis_name="s")
    return pl.kernel(
        functools.partial(_sc_body, cw),
        out_type=jax.ShapeDtypeStruct((_K, _B * _N, cw), jnp.float32),
        mesh=mesh,
        scratch_types=[
            pltpu.VMEM((24, (_B * _N) // _NW), jnp.int32),
            pltpu.VMEM((_K, _CH, 128), jnp.float32),
            pltpu.VMEM((_CH, 128), jnp.float32),
            pltpu.VMEM((_K, _CH, cw), jnp.float32),
            pltpu.SemaphoreType.DMA,
        ],
    )


def _conv_body(diff_ref, xt_ref, wa_ref, wb_ref, mx_ref, pstat_ref):
    t = pl.program_id(1)
    yc = lax.dot_general(xt_ref[0], wb_ref[...], _DN,
                         preferred_element_type=jnp.float32)
    mx = None
    ss = None
    sq = None
    for kk in range(_K):
        yk = lax.dot_general(diff_ref[kk], wa_ref[...], _DN,
                             preferred_element_type=jnp.float32) + yc
        mx = yk if kk == 0 else jnp.maximum(mx, yk)
        ss = yk if kk == 0 else ss + yk
        sq = yk * yk if kk == 0 else sq + yk * yk
    mx_ref[0] = mx

    @pl.when(t == 0)
    def _init():
        pstat_ref[0] = jnp.zeros_like(pstat_ref[0])

    pstat_ref[0, 0:1, :] += jnp.sum(ss, axis=0, keepdims=True)
    pstat_ref[0, 1:2, :] += jnp.sum(sq, axis=0, keepdims=True)


def _run_conv(diff, xtp, wa, wb):
    cw, d = wa.shape
    return pl.pallas_call(
        _conv_body,
        grid=(_B, _NT),
        in_specs=[
            pl.BlockSpec((_K, _T, cw), lambda b, t: (0, b * _NT + t, 0)),
            pl.BlockSpec((1, _T, 128), lambda b, t: (b, t, 0)),
            pl.BlockSpec((cw, d), lambda b, t: (0, 0)),
            pl.BlockSpec((128, d), lambda b, t: (0, 0)),
        ],
        out_specs=[
            pl.BlockSpec((1, _T, d), lambda b, t: (b, t, 0)),
            pl.BlockSpec((1, 8, d), lambda b, t: (b, 0, 0)),
        ],
        out_shape=[
            jax.ShapeDtypeStruct((_B, _N, d), jnp.float32),
            jax.ShapeDtypeStruct((_B, 8, d), jnp.float32),
        ],
    )(diff, xtp, wa, wb)


def _final_body(x1_ref, x2_ref, x3_ref, m4_ref, pstat_ref, g4_ref, b4_ref,
                w5a_ref, w5b_ref, w5c_ref, w5d_ref, g5_ref, b5_ref, out_ref):
    m4, v4 = _bn_scales(pstat_ref, g4_ref, b4_ref, _CNT)
    ssum = None
    ssq = None
    mxs = []
    for b in range(_B):
        x4 = _normalize(m4_ref[b], m4, v4, g4_ref, b4_ref)
        y = (lax.dot_general(x1_ref[b][:, 0:64], w5a_ref[...], _DN,
                             preferred_element_type=jnp.float32)
             + lax.dot_general(x2_ref[b][:, 0:64], w5b_ref[...], _DN,
                               preferred_element_type=jnp.float32)
             + lax.dot_general(x3_ref[b][:, 0:128], w5c_ref[...], _DN,
                               preferred_element_type=jnp.float32)
             + lax.dot_general(x4, w5d_ref[...], _DN,
                               preferred_element_type=jnp.float32))
        s = jnp.sum(y, axis=0, keepdims=True)
        q = jnp.sum(y * y, axis=0, keepdims=True)
        ssum = s if b == 0 else ssum + s
        ssq = q if b == 0 else ssq + q
        mxs.append(jnp.max(y, axis=0, keepdims=True))
    mx = jnp.concatenate(mxs, axis=0)
    cnt = float(_B * _N)
    m5 = ssum / cnt
    v5 = ssq / cnt - m5 * m5
    out_ref[...] = _normalize(mx, m5, v5, g5_ref, b5_ref)


def _run_final(x1p, x2p, x3p, m4, pstat4, g4, b4, w5, g5, b5):
    return pl.pallas_call(
        _final_body,
        out_shape=jax.ShapeDtypeStruct((_B, 1024), jnp.float32),
    )(x1p, x2p, x3p, m4, pstat4, g4, b4,
      w5[0:64], w5[64:128], w5[128:256], w5[256:512], g5, b5)


def kernel(x, W1, g1, b1, W2, g2, b2, W3, g3, b3, W4, g4, b4, W5, g5, b5):
    f32 = jnp.float32
    row = lambda v: v.reshape(1, -1)
    xtp1 = jnp.zeros((_B, _N, 128), f32).at[:, :, :3].set(
        jnp.transpose(x, (0, 2, 1)))

    wa1 = jnp.zeros((16, 64), f32).at[:3].set(W1[:3])
    wb1 = jnp.zeros((128, 64), f32).at[:3].set(W1[3:])
    wa2 = W2[:64]
    wb2 = jnp.zeros((128, 64), f32).at[:64].set(W2[64:])
    wa3 = W3[:64]
    wb3 = jnp.zeros((128, 128), f32).at[:64].set(W3[64:])
    wa4 = W4[:128]
    wb4 = W4[128:]

    idx1 = _run_a1(xtp1)
    diff1 = _make_sc_gather(16)(xtp1.reshape(_B * _N, 128), idx1)
    m1, ps1 = _run_conv(diff1, xtp1, wa1, wb1)

    xtp2, idx2 = _run_a(m1, ps1, row(g1), row(b1))
    diff2 = _make_sc_gather(64)(xtp2.reshape(_B * _N, 128), idx2)
    m2, ps2 = _run_conv(diff2, xtp2, wa2, wb2)

    xtp3, idx3 = _run_a(m2, ps2, row(g2), row(b2))
    diff3 = _make_sc_gather(64)(xtp3.reshape(_B * _N, 128), idx3)
    m3, ps3 = _run_conv(diff3, xtp3, wa3, wb3)

    xtp4, idx4 = _run_a(m3, ps3, row(g3), row(b3))
    diff4 = _make_sc_gather(128)(xtp4.reshape(_B * _N, 128), idx4)
    m4, ps4 = _run_conv(diff4, xtp4, wa4, wb4)

    return _run_final(xtp2, xtp3, xtp4, m4, ps4, row(g4), row(b4),
                      W5, row(g5), row(b5))


# speedup vs baseline: 5.9448x; 1.1570x over previous
import functools

import jax
import jax.numpy as jnp
from jax import lax
from jax.experimental import pallas as pl
from jax.experimental.pallas import tpu as pltpu
from jax.experimental.pallas import tpu_sc as plsc

_K = 20
_KP = 32
_N = 1024
_B = 8
_NW = 32
_NC = 2
_CH = 16
_T = 128
_NT = _N // _T
_EPS = 1e-5
_CNT = float(_B * _N * _K)

_DN = (((1,), (0,)), ((), ()))


def _bn_scales(pstat_ref, g_ref, b_ref, cnt):
    s0 = pstat_ref[0, 0:1, :]
    s1 = pstat_ref[0, 1:2, :]
    for w in range(1, _B):
        s0 = s0 + pstat_ref[w, 0:1, :]
        s1 = s1 + pstat_ref[w, 1:2, :]
    m = s0 / cnt
    v = s1 / cnt - m * m
    return m, v


def _normalize(y, m, v, g_ref, b_ref):
    z = (y - m) / jnp.sqrt(v + _EPS) * g_ref[...] + b_ref[...]
    return jnp.where(z > 0, z, 0.2 * z)


def _knn_core(xtp, b, idx_ref, af_ref):
    n = xtp.shape[0]
    gram = lax.dot_general(xtp, xtp, (((1,), (1,)), ((), ())),
                           preferred_element_type=jnp.float32)
    rows = lax.broadcasted_iota(jnp.int32, (n, n), 0)
    cols = lax.broadcasted_iota(jnp.int32, (n, n), 1)
    colsf = cols.astype(jnp.float32)
    eyef = (rows == cols).astype(jnp.float32)
    xs = jnp.sum(xtp * xtp, axis=1, keepdims=True)
    xsr = lax.dot_general(xs, eyef, (((0,), (0,)), ((), ())),
                          preferred_element_type=jnp.float32,
                          precision=lax.Precision.HIGHEST)
    inner = -2.0 * gram
    pd = (-xs) - inner - xsr
    af_ref[...] = jnp.zeros((n, _KP), jnp.float32)
    nf = jnp.float32(n)
    for kk in range(_K):
        mrow = jnp.max(pd, axis=1, keepdims=True)
        jf = jnp.min(jnp.where(pd == mrow, colsf, nf), axis=1, keepdims=True)
        af_ref[:, kk:kk + 1] = jf
        pd = jnp.where(colsf == jf, -jnp.inf, pd)
    idxt = lax.dot_general(af_ref[...], eyef, (((0,), (0,)), ((), ())),
                           preferred_element_type=jnp.float32,
                           precision=lax.Precision.HIGHEST)
    idx_ref[0] = idxt.astype(jnp.int32) + b * n


def _a1_body(xtp_ref, idx_ref, af_ref):
    _knn_core(xtp_ref[0], pl.program_id(0), idx_ref, af_ref)


def _a_body(c, mx_ref, pstat_ref, g_ref, b_ref, xt_ref, idx_ref, af_ref):
    b = pl.program_id(0)
    m, v = _bn_scales(pstat_ref, g_ref, b_ref, _CNT)
    xtv = _normalize(mx_ref[0], m, v, g_ref, b_ref)
    xt_ref[0, :, 0:c] = xtv
    if c < 128:
        xt_ref[0, :, c:128] = jnp.zeros((_N, 128 - c), jnp.float32)
    _knn_core(xt_ref[0], b, idx_ref, af_ref)


def _run_a1(xtp):
    return pl.pallas_call(
        _a1_body,
        grid=(_B,),
        in_specs=[pl.BlockSpec((1, _N, 128), lambda b: (b, 0, 0))],
        out_specs=pl.BlockSpec((1, _KP, _N), lambda b: (b, 0, 0)),
        out_shape=jax.ShapeDtypeStruct((_B, _KP, _N), jnp.int32),
        scratch_shapes=[pltpu.VMEM((_N, _KP), jnp.float32)],
    )(xtp)


def _run_a(mx, pstat, g, b):
    c = mx.shape[2]
    return pl.pallas_call(
        functools.partial(_a_body, c),
        grid=(_B,),
        in_specs=[
            pl.BlockSpec((1, _N, c), lambda b: (b, 0, 0)),
            pl.BlockSpec((_B, 8, c), lambda b: (0, 0, 0)),
            pl.BlockSpec((1, c), lambda b: (0, 0)),
            pl.BlockSpec((1, c), lambda b: (0, 0)),
        ],
        out_specs=[
            pl.BlockSpec((1, _N, 128), lambda b: (b, 0, 0)),
            pl.BlockSpec((1, _KP, _N), lambda b: (b, 0, 0)),
        ],
        out_shape=[
            jax.ShapeDtypeStruct((_B, _N, 128), jnp.float32),
            jax.ShapeDtypeStruct((_B, _KP, _N), jnp.int32),
        ],
        scratch_shapes=[pltpu.VMEM((_N, _KP), jnp.float32)],
    )(mx, pstat, g, b)


def _sc_body(cw, xt_hbm, idx_hbm, diff_hbm, idxb, rows, cb, diffb, gsem):
    seg = cw // 16
    ppw = (_B * _N) // _NW
    nch = ppw // _CH
    wid = lax.axis_index("s") * _NC + lax.axis_index("c")
    base = wid * ppw
    bb = base // _N
    nb = base - bb * _N
    pltpu.sync_copy(idx_hbm.at[bb, pl.ds(0, 24), pl.ds(nb, ppw)], idxb)

    def chunk(ch, carry):
        p0 = base + ch * _CH
        pltpu.sync_copy(xt_hbm.at[pl.ds(p0, _CH)], cb)
        handles = [pltpu.async_copy(
            xt_hbm.at[idxb.at[kk, pl.ds(ch * _CH, _CH)]], rows.at[kk], gsem)
            for kk in range(_K)]
        for h in handles:
            h.wait()

        def point(n, c2):
            for sg in range(seg):
                sl = pl.ds(sg * 16, 16)
                cv = cb[n, sl]
                for kk in range(_K):
                    diffb[kk, n, sl] = rows[kk, n, sl] - cv
            return c2

        lax.fori_loop(0, _CH, point, 0)
        pltpu.sync_copy(diffb, diff_hbm.at[pl.ds(0, _K), pl.ds(p0, _CH)])
        return carry

    lax.fori_loop(0, nch, chunk, 0)


@functools.cache
def _make_sc_gather(cw):
    mesh = plsc.VectorSubcoreMesh(core_axis_name="c", subcore_axis_name="s")
    return pl.kernel(
        functools.partial(_sc_body, cw),
        out_type=jax.ShapeDtypeStruct((_K, _B * _N, cw), jnp.float32),
        mesh=mesh,
        scratch_types=[
            pltpu.VMEM((24, (_B * _N) // _NW), jnp.int32),
            pltpu.VMEM((_K, _CH, 128), jnp.float32),
            pltpu.VMEM((_CH, 128), jnp.float32),
            pltpu.VMEM((_K, _CH, cw), jnp.float32),
            pltpu.SemaphoreType.DMA,
        ],
    )


def _conv_body(diff_ref, xt_ref, wa_ref, wb_ref, mx_ref, pstat_ref):
    t = pl.program_id(1)
    yc = lax.dot_general(xt_ref[0], wb_ref[...], _DN,
                         preferred_element_type=jnp.float32)
    mx = None
    ss = None
    sq = None
    for kk in range(_K):
        yk = lax.dot_general(diff_ref[kk], wa_ref[...], _DN,
                             preferred_element_type=jnp.float32) + yc
        mx = yk if kk == 0 else jnp.maximum(mx, yk)
        ss = yk if kk == 0 else ss + yk
        sq = yk * yk if kk == 0 else sq + yk * yk
    mx_ref[0] = mx

    @pl.when(t == 0)
    def _init():
        pstat_ref[0] = jnp.zeros_like(pstat_ref[0])

    pstat_ref[0, 0:1, :] += jnp.sum(ss, axis=0, keepdims=True)
    pstat_ref[0, 1:2, :] += jnp.sum(sq, axis=0, keepdims=True)


def _run_conv(diff, xtp, wa, wb):
    cw, d = wa.shape
    return pl.pallas_call(
        _conv_body,
        grid=(_B, _NT),
        in_specs=[
            pl.BlockSpec((_K, _T, cw), lambda b, t: (0, b * _NT + t, 0)),
            pl.BlockSpec((1, _T, 128), lambda b, t: (b, t, 0)),
            pl.BlockSpec((cw, d), lambda b, t: (0, 0)),
            pl.BlockSpec((128, d), lambda b, t: (0, 0)),
        ],
        out_specs=[
            pl.BlockSpec((1, _T, d), lambda b, t: (b, t, 0)),
            pl.BlockSpec((1, 8, d), lambda b, t: (b, 0, 0)),
        ],
        out_shape=[
            jax.ShapeDtypeStruct((_B, _N, d), jnp.float32),
            jax.ShapeDtypeStruct((_B, 8, d), jnp.float32),
        ],
    )(diff, xtp, wa, wb)


def _final_body(x1_ref, x2_ref, x3_ref, m4_ref, pstat_ref, g4_ref, b4_ref,
                w5a_ref, w5b_ref, w5c_ref, w5d_ref, g5_ref, b5_ref, out_ref):
    m4, v4 = _bn_scales(pstat_ref, g4_ref, b4_ref, _CNT)
    ssum = None
    ssq = None
    mxs = []
    for b in range(_B):
        x4 = _normalize(m4_ref[b], m4, v4, g4_ref, b4_ref)
        y = (lax.dot_general(x1_ref[b][:, 0:64], w5a_ref[...], _DN,
                             preferred_element_type=jnp.float32)
             + lax.dot_general(x2_ref[b][:, 0:64], w5b_ref[...], _DN,
                               preferred_element_type=jnp.float32)
             + lax.dot_general(x3_ref[b][:, 0:128], w5c_ref[...], _DN,
                               preferred_element_type=jnp.float32)
             + lax.dot_general(x4, w5d_ref[...], _DN,
                               preferred_element_type=jnp.float32))
        s = jnp.sum(y, axis=0, keepdims=True)
        q = jnp.sum(y * y, axis=0, keepdims=True)
        ssum = s if b == 0 else ssum + s
        ssq = q if b == 0 else ssq + q
        mxs.append(jnp.max(y, axis=0, keepdims=True))
    mx = jnp.concatenate(mxs, axis=0)
    cnt = float(_B * _N)
    m5 = ssum / cnt
    v5 = ssq / cnt - m5 * m5
    out_ref[...] = _normalize(mx, m5, v5, g5_ref, b5_ref)


def _run_final(x1p, x2p, x3p, m4, pstat4, g4, b4, w5, g5, b5):
    return pl.pallas_call(
        _final_body,
        out_shape=jax.ShapeDtypeStruct((_B, 1024), jnp.float32),
    )(x1p, x2p, x3p, m4, pstat4, g4, b4,
      w5[0:64], w5[64:128], w5[128:256], w5[256:512], g5, b5)


def kernel(x, W1, g1, b1, W2, g2, b2, W3, g3, b3, W4, g4, b4, W5, g5, b5):
    f32 = jnp.float32
    row = lambda v: v.reshape(1, -1)
    xtp1 = jnp.zeros((_B, _N, 128), f32).at[:, :, :3].set(
        jnp.transpose(x, (0, 2, 1)))

    wa1 = jnp.zeros((16, 64), f32).at[:3].set(W1[:3])
    wb1 = jnp.zeros((128, 64), f32).at[:3].set(W1[3:])
    wa2 = W2[:64]
    wb2 = jnp.zeros((128, 64), f32).at[:64].set(W2[64:])
    wa3 = W3[:64]
    wb3 = jnp.zeros((128, 128), f32).at[:64].set(W3[64:])
    wa4 = W4[:128]
    wb4 = W4[128:]

    idx1 = _run_a1(xtp1)
    diff1 = _make_sc_gather(16)(xtp1.reshape(_B * _N, 128), idx1)
    m1, ps1 = _run_conv(diff1, xtp1, wa1, wb1)

    xtp2, idx2 = _run_a(m1, ps1, row(g1), row(b1))
    diff2 = _make_sc_gather(64)(xtp2.reshape(_B * _N, 128), idx2)
    m2, ps2 = _run_conv(diff2, xtp2, wa2, wb2)

    xtp3, idx3 = _run_a(m2, ps2, row(g2), row(b2))
    diff3 = _make_sc_gather(64)(xtp3.reshape(_B * _N, 128), idx3)
    m3, ps3 = _run_conv(diff3, xtp3, wa3, wb3)

    xtp4, idx4 = _run_a(m3, ps3, row(g3), row(b3))
    diff4 = _make_sc_gather(128)(xtp4.reshape(_B * _N, 128), idx4)
    m4, ps4 = _run_conv(diff4, xtp4, wa4, wb4)

    return _run_final(xtp2, xtp3, xtp4, m4, ps4, row(g4), row(b4),
                      W5, row(g5), row(b5))


# speedup vs baseline: 6.1609x; 1.0363x over previous
import functools

import jax
import jax.numpy as jnp
from jax import lax
from jax.experimental import pallas as pl
from jax.experimental.pallas import tpu as pltpu
from jax.experimental.pallas import tpu_sc as plsc

_K = 20
_KP = 32
_N = 1024
_B = 8
_NW = 32
_NC = 2
_CH = 16
_T = 128
_NT = _N // _T
_EPS = 1e-5
_CNT = float(_B * _N * _K)

_DN = (((1,), (0,)), ((), ()))


def _bn_scales(pstat_ref, g_ref, b_ref, cnt):
    s0 = pstat_ref[0, 0:1, :]
    s1 = pstat_ref[0, 1:2, :]
    for w in range(1, _B):
        s0 = s0 + pstat_ref[w, 0:1, :]
        s1 = s1 + pstat_ref[w, 1:2, :]
    m = s0 / cnt
    v = s1 / cnt - m * m
    return m, v


def _normalize(y, m, v, g_ref, b_ref):
    z = (y - m) / jnp.sqrt(v + _EPS) * g_ref[...] + b_ref[...]
    return jnp.where(z > 0, z, 0.2 * z)


def _knn_core(xtp, b, idx_ref, af_ref):
    n = xtp.shape[0]
    gram = lax.dot_general(xtp, xtp, (((1,), (1,)), ((), ())),
                           preferred_element_type=jnp.float32)
    rows = lax.broadcasted_iota(jnp.int32, (n, n), 0)
    cols = lax.broadcasted_iota(jnp.int32, (n, n), 1)
    colsf = cols.astype(jnp.float32)
    eyef = (rows == cols).astype(jnp.float32)
    xs = jnp.sum(xtp * xtp, axis=1, keepdims=True)
    xsr = lax.dot_general(xs, eyef, (((0,), (0,)), ((), ())),
                          preferred_element_type=jnp.float32,
                          precision=lax.Precision.HIGHEST)
    inner = -2.0 * gram
    pd = (-xs) - inner - xsr
    af_ref[...] = jnp.zeros((n, _KP), jnp.float32)
    nf = jnp.float32(n)
    for kk in range(_K):
        mrow = jnp.max(pd, axis=1, keepdims=True)
        jf = jnp.min(jnp.where(pd == mrow, colsf, nf), axis=1, keepdims=True)
        af_ref[:, kk:kk + 1] = jf
        pd = jnp.where(colsf == jf, -jnp.inf, pd)
    idxt = lax.dot_general(af_ref[...], eyef, (((0,), (0,)), ((), ())),
                           preferred_element_type=jnp.float32,
                           precision=lax.Precision.HIGHEST)
    idx_ref[0] = idxt.astype(jnp.int32) + b * n


def _a1_body(xtp_ref, idx_ref, af_ref):
    _knn_core(xtp_ref[0], pl.program_id(0), idx_ref, af_ref)


def _a_body(c, mx_ref, pstat_ref, g_ref, b_ref, xt_ref, idx_ref, af_ref):
    b = pl.program_id(0)
    m, v = _bn_scales(pstat_ref, g_ref, b_ref, _CNT)
    xtv = _normalize(mx_ref[0], m, v, g_ref, b_ref)
    xt_ref[0, :, 0:c] = xtv
    if c < 128:
        xt_ref[0, :, c:128] = jnp.zeros((_N, 128 - c), jnp.float32)
    _knn_core(xt_ref[0], b, idx_ref, af_ref)


def _run_a1(xtp):
    return pl.pallas_call(
        _a1_body,
        grid=(_B,),
        in_specs=[pl.BlockSpec((1, _N, 128), lambda b: (b, 0, 0))],
        out_specs=pl.BlockSpec((1, _KP, _N), lambda b: (b, 0, 0)),
        out_shape=jax.ShapeDtypeStruct((_B, _KP, _N), jnp.int32),
        scratch_shapes=[pltpu.VMEM((_N, _KP), jnp.float32)],
    )(xtp)


def _run_a(mx, pstat, g, b):
    c = mx.shape[2]
    return pl.pallas_call(
        functools.partial(_a_body, c),
        grid=(_B,),
        in_specs=[
            pl.BlockSpec((1, _N, c), lambda b: (b, 0, 0)),
            pl.BlockSpec((_B, 8, c), lambda b: (0, 0, 0)),
            pl.BlockSpec((1, c), lambda b: (0, 0)),
            pl.BlockSpec((1, c), lambda b: (0, 0)),
        ],
        out_specs=[
            pl.BlockSpec((1, _N, 128), lambda b: (b, 0, 0)),
            pl.BlockSpec((1, _KP, _N), lambda b: (b, 0, 0)),
        ],
        out_shape=[
            jax.ShapeDtypeStruct((_B, _N, 128), jnp.float32),
            jax.ShapeDtypeStruct((_B, _KP, _N), jnp.int32),
        ],
        scratch_shapes=[pltpu.VMEM((_N, _KP), jnp.float32)],
    )(mx, pstat, g, b)


def _sc_body(cw, xt_hbm, idx_hbm, diff_hbm, *scr):
    if cw == 128:
        idxb, rows, cb, sem0, sem1 = scr
        diffb = None
    else:
        idxb, rows, cb, diffb, sem0, sem1 = scr
    sems = (sem0, sem1)
    ch_sz = _CH if cw == 128 else 8
    seg = cw // 16
    ppw = (_B * _N) // _NW
    nch = ppw // ch_sz
    wid = lax.axis_index("s") * _NC + lax.axis_index("c")
    base = wid * ppw
    bb = base // _N
    nb = base - bb * _N
    pltpu.sync_copy(idx_hbm.at[bb, pl.ds(0, 24), pl.ds(nb, ppw)], idxb)

    def chunk_pair(i2, carry):
        c0 = i2 * 2
        hs = []
        for par in (0, 1):
            ch = c0 + par
            h_cb = pltpu.async_copy(
                xt_hbm.at[pl.ds(base + ch * ch_sz, ch_sz)], cb.at[par],
                sems[par])
            h_g = [pltpu.async_copy(
                xt_hbm.at[idxb.at[kk, pl.ds(ch * ch_sz, ch_sz)]],
                rows.at[par, kk], sems[par]) for kk in range(_K)]
            hs.append((h_cb, h_g))
        for par in (0, 1):
            ch = c0 + par
            h_cb, h_g = hs[par]
            h_cb.wait()
            for h in h_g:
                h.wait()

            def point(n, c2):
                for sg in range(seg):
                    sl = pl.ds(sg * 16, 16)
                    cv = cb[par, n, sl]
                    for kk in range(_K):
                        if cw == 128:
                            rows[par, kk, n, sl] = rows[par, kk, n, sl] - cv
                        else:
                            diffb[kk, n, sl] = rows[par, kk, n, sl] - cv
                return c2

            lax.fori_loop(0, ch_sz, point, 0)
            src = rows.at[par] if cw == 128 else diffb
            pltpu.sync_copy(
                src,
                diff_hbm.at[pl.ds(0, _K), pl.ds(base + ch * ch_sz, ch_sz)])
        return carry

    lax.fori_loop(0, nch // 2, chunk_pair, 0)


@functools.cache
def _make_sc_gather(cw):
    mesh = plsc.VectorSubcoreMesh(core_axis_name="c", subcore_axis_name="s")
    ch_sz = _CH if cw == 128 else 8
    scratch = [
        pltpu.VMEM((24, (_B * _N) // _NW), jnp.int32),
        pltpu.VMEM((2, _K, ch_sz, 128), jnp.float32),
        pltpu.VMEM((2, ch_sz, 128), jnp.float32),
    ]
    if cw != 128:
        scratch.append(pltpu.VMEM((_K, ch_sz, cw), jnp.float32))
    scratch += [pltpu.SemaphoreType.DMA, pltpu.SemaphoreType.DMA]
    return pl.kernel(
        functools.partial(_sc_body, cw),
        out_type=jax.ShapeDtypeStruct((_K, _B * _N, cw), jnp.float32),
        mesh=mesh,
        scratch_types=scratch,
    )


def _conv_body(diff_ref, xt_ref, wa_ref, wb_ref, mx_ref, pstat_ref):
    t = pl.program_id(1)
    yc = lax.dot_general(xt_ref[0], wb_ref[...], _DN,
                         preferred_element_type=jnp.float32)
    mx = None
    ss = None
    sq = None
    for kk in range(_K):
        yk = lax.dot_general(diff_ref[kk], wa_ref[...], _DN,
                             preferred_element_type=jnp.float32) + yc
        mx = yk if kk == 0 else jnp.maximum(mx, yk)
        ss = yk if kk == 0 else ss + yk
        sq = yk * yk if kk == 0 else sq + yk * yk
    mx_ref[0] = mx

    @pl.when(t == 0)
    def _init():
        pstat_ref[0] = jnp.zeros_like(pstat_ref[0])

    pstat_ref[0, 0:1, :] += jnp.sum(ss, axis=0, keepdims=True)
    pstat_ref[0, 1:2, :] += jnp.sum(sq, axis=0, keepdims=True)


def _run_conv(diff, xtp, wa, wb):
    cw, d = wa.shape
    return pl.pallas_call(
        _conv_body,
        grid=(_B, _NT),
        in_specs=[
            pl.BlockSpec((_K, _T, cw), lambda b, t: (0, b * _NT + t, 0)),
            pl.BlockSpec((1, _T, 128), lambda b, t: (b, t, 0)),
            pl.BlockSpec((cw, d), lambda b, t: (0, 0)),
            pl.BlockSpec((128, d), lambda b, t: (0, 0)),
        ],
        out_specs=[
            pl.BlockSpec((1, _T, d), lambda b, t: (b, t, 0)),
            pl.BlockSpec((1, 8, d), lambda b, t: (b, 0, 0)),
        ],
        out_shape=[
            jax.ShapeDtypeStruct((_B, _N, d), jnp.float32),
            jax.ShapeDtypeStruct((_B, 8, d), jnp.float32),
        ],
    )(diff, xtp, wa, wb)


def _final_body(x1_ref, x2_ref, x3_ref, m4_ref, pstat_ref, g4_ref, b4_ref,
                w5a_ref, w5b_ref, w5c_ref, w5d_ref, g5_ref, b5_ref, out_ref):
    m4, v4 = _bn_scales(pstat_ref, g4_ref, b4_ref, _CNT)
    ssum = None
    ssq = None
    mxs = []
    for b in range(_B):
        x4 = _normalize(m4_ref[b], m4, v4, g4_ref, b4_ref)
        y = (lax.dot_general(x1_ref[b][:, 0:64], w5a_ref[...], _DN,
                             preferred_element_type=jnp.float32)
             + lax.dot_general(x2_ref[b][:, 0:64], w5b_ref[...], _DN,
                               preferred_element_type=jnp.float32)
             + lax.dot_general(x3_ref[b][:, 0:128], w5c_ref[...], _DN,
                               preferred_element_type=jnp.float32)
             + lax.dot_general(x4, w5d_ref[...], _DN,
                               preferred_element_type=jnp.float32))
        s = jnp.sum(y, axis=0, keepdims=True)
        q = jnp.sum(y * y, axis=0, keepdims=True)
        ssum = s if b == 0 else ssum + s
        ssq = q if b == 0 else ssq + q
        mxs.append(jnp.max(y, axis=0, keepdims=True))
    mx = jnp.concatenate(mxs, axis=0)
    cnt = float(_B * _N)
    m5 = ssum / cnt
    v5 = ssq / cnt - m5 * m5
    out_ref[...] = _normalize(mx, m5, v5, g5_ref, b5_ref)


def _run_final(x1p, x2p, x3p, m4, pstat4, g4, b4, w5, g5, b5):
    return pl.pallas_call(
        _final_body,
        out_shape=jax.ShapeDtypeStruct((_B, 1024), jnp.float32),
    )(x1p, x2p, x3p, m4, pstat4, g4, b4,
      w5[0:64], w5[64:128], w5[128:256], w5[256:512], g5, b5)


def kernel(x, W1, g1, b1, W2, g2, b2, W3, g3, b3, W4, g4, b4, W5, g5, b5):
    f32 = jnp.float32
    row = lambda v: v.reshape(1, -1)
    xtp1 = jnp.zeros((_B, _N, 128), f32).at[:, :, :3].set(
        jnp.transpose(x, (0, 2, 1)))

    wa1 = jnp.zeros((16, 64), f32).at[:3].set(W1[:3])
    wb1 = jnp.zeros((128, 64), f32).at[:3].set(W1[3:])
    wa2 = W2[:64]
    wb2 = jnp.zeros((128, 64), f32).at[:64].set(W2[64:])
    wa3 = W3[:64]
    wb3 = jnp.zeros((128, 128), f32).at[:64].set(W3[64:])
    wa4 = W4[:128]
    wb4 = W4[128:]

    idx1 = _run_a1(xtp1)
    diff1 = _make_sc_gather(16)(xtp1.reshape(_B * _N, 128), idx1)
    m1, ps1 = _run_conv(diff1, xtp1, wa1, wb1)

    xtp2, idx2 = _run_a(m1, ps1, row(g1), row(b1))
    diff2 = _make_sc_gather(64)(xtp2.reshape(_B * _N, 128), idx2)
    m2, ps2 = _run_conv(diff2, xtp2, wa2, wb2)

    xtp3, idx3 = _run_a(m2, ps2, row(g2), row(b2))
    diff3 = _make_sc_gather(64)(xtp3.reshape(_B * _N, 128), idx3)
    m3, ps3 = _run_conv(diff3, xtp3, wa3, wb3)

    xtp4, idx4 = _run_a(m3, ps3, row(g3), row(b3))
    diff4 = _make_sc_gather(128)(xtp4.reshape(_B * _N, 128), idx4)
    m4, ps4 = _run_conv(diff4, xtp4, wa4, wb4)

    return _run_final(xtp2, xtp3, xtp4, m4, ps4, row(g4), row(b4),
                      W5, row(g5), row(b5))


# speedup vs baseline: 6.5075x; 1.0563x over previous
import functools

import jax
import jax.numpy as jnp
from jax import lax
from jax.experimental import pallas as pl
from jax.experimental.pallas import tpu as pltpu
from jax.experimental.pallas import tpu_sc as plsc

_K = 20
_KP = 32
_N = 1024
_B = 8
_NW = 32
_NC = 2
_CH = 16
_T = 256
_NT = _N // _T
_EPS = 1e-5
_CNT = float(_B * _N * _K)

_DN = (((1,), (0,)), ((), ()))


def _bn_scales(pstat_ref, g_ref, b_ref, cnt):
    s0 = pstat_ref[0, 0:1, :]
    s1 = pstat_ref[0, 1:2, :]
    for w in range(1, _B):
        s0 = s0 + pstat_ref[w, 0:1, :]
        s1 = s1 + pstat_ref[w, 1:2, :]
    m = s0 / cnt
    v = s1 / cnt - m * m
    return m, v


def _normalize(y, m, v, g_ref, b_ref):
    z = (y - m) / jnp.sqrt(v + _EPS) * g_ref[...] + b_ref[...]
    return jnp.where(z > 0, z, 0.2 * z)


def _knn_core(xtp, b, idx_ref, af_ref):
    n = xtp.shape[0]
    gram = lax.dot_general(xtp, xtp, (((1,), (1,)), ((), ())),
                           preferred_element_type=jnp.float32)
    rows = lax.broadcasted_iota(jnp.int32, (n, n), 0)
    cols = lax.broadcasted_iota(jnp.int32, (n, n), 1)
    colsf = cols.astype(jnp.float32)
    eyef = (rows == cols).astype(jnp.float32)
    xs = jnp.sum(xtp * xtp, axis=1, keepdims=True)
    xsr = lax.dot_general(xs, eyef, (((0,), (0,)), ((), ())),
                          preferred_element_type=jnp.float32,
                          precision=lax.Precision.HIGHEST)
    inner = -2.0 * gram
    pd = (-xs) - inner - xsr
    af_ref[...] = jnp.zeros((n, _KP), jnp.float32)
    nf = jnp.float32(n)
    for kk in range(_K):
        mrow = jnp.max(pd, axis=1, keepdims=True)
        jf = jnp.min(jnp.where(pd == mrow, colsf, nf), axis=1, keepdims=True)
        af_ref[:, kk:kk + 1] = jf
        pd = jnp.where(colsf == jf, -jnp.inf, pd)
    idxt = lax.dot_general(af_ref[...], eyef, (((0,), (0,)), ((), ())),
                           preferred_element_type=jnp.float32,
                           precision=lax.Precision.HIGHEST)
    idx_ref[0] = idxt.astype(jnp.int32) + b * n


def _a1_body(xtp_ref, idx_ref, af_ref):
    _knn_core(xtp_ref[0], pl.program_id(0), idx_ref, af_ref)


def _a_body(c, mx_ref, pstat_ref, g_ref, b_ref, xt_ref, idx_ref, af_ref):
    b = pl.program_id(0)
    m, v = _bn_scales(pstat_ref, g_ref, b_ref, _CNT)
    xtv = _normalize(mx_ref[0], m, v, g_ref, b_ref)
    xt_ref[0, :, 0:c] = xtv
    if c < 128:
        xt_ref[0, :, c:128] = jnp.zeros((_N, 128 - c), jnp.float32)
    _knn_core(xt_ref[0], b, idx_ref, af_ref)


def _run_a1(xtp):
    return pl.pallas_call(
        _a1_body,
        grid=(_B,),
        in_specs=[pl.BlockSpec((1, _N, 128), lambda b: (b, 0, 0))],
        out_specs=pl.BlockSpec((1, _KP, _N), lambda b: (b, 0, 0)),
        out_shape=jax.ShapeDtypeStruct((_B, _KP, _N), jnp.int32),
        scratch_shapes=[pltpu.VMEM((_N, _KP), jnp.float32)],
    )(xtp)


def _run_a(mx, pstat, g, b):
    c = mx.shape[2]
    return pl.pallas_call(
        functools.partial(_a_body, c),
        grid=(_B,),
        in_specs=[
            pl.BlockSpec((1, _N, c), lambda b: (b, 0, 0)),
            pl.BlockSpec((_B, 8, c), lambda b: (0, 0, 0)),
            pl.BlockSpec((1, c), lambda b: (0, 0)),
            pl.BlockSpec((1, c), lambda b: (0, 0)),
        ],
        out_specs=[
            pl.BlockSpec((1, _N, 128), lambda b: (b, 0, 0)),
            pl.BlockSpec((1, _KP, _N), lambda b: (b, 0, 0)),
        ],
        out_shape=[
            jax.ShapeDtypeStruct((_B, _N, 128), jnp.float32),
            jax.ShapeDtypeStruct((_B, _KP, _N), jnp.int32),
        ],
        scratch_shapes=[pltpu.VMEM((_N, _KP), jnp.float32)],
    )(mx, pstat, g, b)


def _sc_body(cw, xt_hbm, idx_hbm, diff_hbm, *scr):
    if cw == 128:
        idxb, rows, cb, sem0, sem1 = scr
        diffb = None
    else:
        idxb, rows, cb, diffb, sem0, sem1 = scr
    sems = (sem0, sem1)
    ch_sz = _CH if cw == 128 else 8
    seg = cw // 16
    ppw = (_B * _N) // _NW
    nch = ppw // ch_sz
    wid = lax.axis_index("s") * _NC + lax.axis_index("c")
    base = wid * ppw
    bb = base // _N
    nb = base - bb * _N
    pltpu.sync_copy(idx_hbm.at[bb, pl.ds(0, 24), pl.ds(nb, ppw)], idxb)

    def chunk_pair(i2, carry):
        c0 = i2 * 2
        hs = []
        for par in (0, 1):
            ch = c0 + par
            h_cb = pltpu.async_copy(
                xt_hbm.at[pl.ds(base + ch * ch_sz, ch_sz)], cb.at[par],
                sems[par])
            h_g = [pltpu.async_copy(
                xt_hbm.at[idxb.at[kk, pl.ds(ch * ch_sz, ch_sz)]],
                rows.at[par, kk], sems[par]) for kk in range(_K)]
            hs.append((h_cb, h_g))
        for par in (0, 1):
            ch = c0 + par
            h_cb, h_g = hs[par]
            h_cb.wait()
            for h in h_g:
                h.wait()

            def point(n, c2):
                for sg in range(seg):
                    sl = pl.ds(sg * 16, 16)
                    cv = cb[par, n, sl]
                    for kk in range(_K):
                        if cw == 128:
                            rows[par, kk, n, sl] = rows[par, kk, n, sl] - cv
                        else:
                            diffb[kk, n, sl] = rows[par, kk, n, sl] - cv
                return c2

            lax.fori_loop(0, ch_sz, point, 0)
            src = rows.at[par] if cw == 128 else diffb
            pltpu.sync_copy(
                src,
                diff_hbm.at[pl.ds(0, _K), pl.ds(base + ch * ch_sz, ch_sz)])
        return carry

    lax.fori_loop(0, nch // 2, chunk_pair, 0)


@functools.cache
def _make_sc_gather(cw):
    mesh = plsc.VectorSubcoreMesh(core_axis_name="c", subcore_axis_name="s")
    ch_sz = _CH if cw == 128 else 8
    scratch = [
        pltpu.VMEM((24, (_B * _N) // _NW), jnp.int32),
        pltpu.VMEM((2, _K, ch_sz, 128), jnp.float32),
        pltpu.VMEM((2, ch_sz, 128), jnp.float32),
    ]
    if cw != 128:
        scratch.append(pltpu.VMEM((_K, ch_sz, cw), jnp.float32))
    scratch += [pltpu.SemaphoreType.DMA, pltpu.SemaphoreType.DMA]
    return pl.kernel(
        functools.partial(_sc_body, cw),
        out_type=jax.ShapeDtypeStruct((_K, _B * _N, cw), jnp.float32),
        mesh=mesh,
        scratch_types=scratch,
    )


def _conv_body(diff_ref, xt_ref, wa_ref, wb_ref, mx_ref, pstat_ref):
    t = pl.program_id(1)
    yc = lax.dot_general(xt_ref[0], wb_ref[...], _DN,
                         preferred_element_type=jnp.float32)
    mx = None
    ss = None
    sq = None
    for kk in range(_K):
        yk = lax.dot_general(diff_ref[kk], wa_ref[...], _DN,
                             preferred_element_type=jnp.float32) + yc
        mx = yk if kk == 0 else jnp.maximum(mx, yk)
        ss = yk if kk == 0 else ss + yk
        sq = yk * yk if kk == 0 else sq + yk * yk
    mx_ref[0] = mx

    @pl.when(t == 0)
    def _init():
        pstat_ref[0] = jnp.zeros_like(pstat_ref[0])

    pstat_ref[0, 0:1, :] += jnp.sum(ss, axis=0, keepdims=True)
    pstat_ref[0, 1:2, :] += jnp.sum(sq, axis=0, keepdims=True)


def _run_conv(diff, xtp, wa, wb):
    cw, d = wa.shape
    return pl.pallas_call(
        _conv_body,
        grid=(_B, _NT),
        in_specs=[
            pl.BlockSpec((_K, _T, cw), lambda b, t: (0, b * _NT + t, 0)),
            pl.BlockSpec((1, _T, 128), lambda b, t: (b, t, 0)),
            pl.BlockSpec((cw, d), lambda b, t: (0, 0)),
            pl.BlockSpec((128, d), lambda b, t: (0, 0)),
        ],
        out_specs=[
            pl.BlockSpec((1, _T, d), lambda b, t: (b, t, 0)),
            pl.BlockSpec((1, 8, d), lambda b, t: (b, 0, 0)),
        ],
        out_shape=[
            jax.ShapeDtypeStruct((_B, _N, d), jnp.float32),
            jax.ShapeDtypeStruct((_B, 8, d), jnp.float32),
        ],
    )(diff, xtp, wa, wb)


def _final_body(x1_ref, x2_ref, x3_ref, m4_ref, pstat_ref, g4_ref, b4_ref,
                w5a_ref, w5b_ref, w5c_ref, w5d_ref, g5_ref, b5_ref, out_ref):
    m4, v4 = _bn_scales(pstat_ref, g4_ref, b4_ref, _CNT)
    ssum = None
    ssq = None
    mxs = []
    for b in range(_B):
        x4 = _normalize(m4_ref[b], m4, v4, g4_ref, b4_ref)
        y = (lax.dot_general(x1_ref[b][:, 0:64], w5a_ref[...], _DN,
                             preferred_element_type=jnp.float32)
             + lax.dot_general(x2_ref[b][:, 0:64], w5b_ref[...], _DN,
                               preferred_element_type=jnp.float32)
             + lax.dot_general(x3_ref[b][:, 0:128], w5c_ref[...], _DN,
                               preferred_element_type=jnp.float32)
             + lax.dot_general(x4, w5d_ref[...], _DN,
                               preferred_element_type=jnp.float32))
        s = jnp.sum(y, axis=0, keepdims=True)
        q = jnp.sum(y * y, axis=0, keepdims=True)
        ssum = s if b == 0 else ssum + s
        ssq = q if b == 0 else ssq + q
        mxs.append(jnp.max(y, axis=0, keepdims=True))
    mx = jnp.concatenate(mxs, axis=0)
    cnt = float(_B * _N)
    m5 = ssum / cnt
    v5 = ssq / cnt - m5 * m5
    out_ref[...] = _normalize(mx, m5, v5, g5_ref, b5_ref)


def _run_final(x1p, x2p, x3p, m4, pstat4, g4, b4, w5, g5, b5):
    return pl.pallas_call(
        _final_body,
        out_shape=jax.ShapeDtypeStruct((_B, 1024), jnp.float32),
    )(x1p, x2p, x3p, m4, pstat4, g4, b4,
      w5[0:64], w5[64:128], w5[128:256], w5[256:512], g5, b5)


def kernel(x, W1, g1, b1, W2, g2, b2, W3, g3, b3, W4, g4, b4, W5, g5, b5):
    f32 = jnp.float32
    row = lambda v: v.reshape(1, -1)
    xtp1 = jnp.zeros((_B, _N, 128), f32).at[:, :, :3].set(
        jnp.transpose(x, (0, 2, 1)))

    wa1 = jnp.zeros((16, 64), f32).at[:3].set(W1[:3])
    wb1 = jnp.zeros((128, 64), f32).at[:3].set(W1[3:])
    wa2 = W2[:64]
    wb2 = jnp.zeros((128, 64), f32).at[:64].set(W2[64:])
    wa3 = W3[:64]
    wb3 = jnp.zeros((128, 128), f32).at[:64].set(W3[64:])
    wa4 = W4[:128]
    wb4 = W4[128:]

    idx1 = _run_a1(xtp1)
    diff1 = _make_sc_gather(16)(xtp1.reshape(_B * _N, 128), idx1)
    m1, ps1 = _run_conv(diff1, xtp1, wa1, wb1)

    xtp2, idx2 = _run_a(m1, ps1, row(g1), row(b1))
    diff2 = _make_sc_gather(64)(xtp2.reshape(_B * _N, 128), idx2)
    m2, ps2 = _run_conv(diff2, xtp2, wa2, wb2)

    xtp3, idx3 = _run_a(m2, ps2, row(g2), row(b2))
    diff3 = _make_sc_gather(64)(xtp3.reshape(_B * _N, 128), idx3)
    m3, ps3 = _run_conv(diff3, xtp3, wa3, wb3)

    xtp4, idx4 = _run_a(m3, ps3, row(g3), row(b3))
    diff4 = _make_sc_gather(128)(xtp4.reshape(_B * _N, 128), idx4)
    m4, ps4 = _run_conv(diff4, xtp4, wa4, wb4)

    return _run_final(xtp2, xtp3, xtp4, m4, ps4, row(g4), row(b4),
                      W5, row(g5), row(b5))


# speedup vs baseline: 6.6799x; 1.0265x over previous
import functools

import jax
import jax.numpy as jnp
from jax import lax
from jax.experimental import pallas as pl
from jax.experimental.pallas import tpu as pltpu
from jax.experimental.pallas import tpu_sc as plsc

_K = 20
_KP = 32
_N = 1024
_B = 8
_NW = 32
_NC = 2
_CH = 16
_T = 512
_NT = _N // _T
_EPS = 1e-5
_CNT = float(_B * _N * _K)

_DN = (((1,), (0,)), ((), ()))


def _bn_scales(pstat_ref, g_ref, b_ref, cnt):
    s0 = pstat_ref[0, 0:1, :]
    s1 = pstat_ref[0, 1:2, :]
    for w in range(1, _B):
        s0 = s0 + pstat_ref[w, 0:1, :]
        s1 = s1 + pstat_ref[w, 1:2, :]
    m = s0 / cnt
    v = s1 / cnt - m * m
    return m, v


def _normalize(y, m, v, g_ref, b_ref):
    z = (y - m) / jnp.sqrt(v + _EPS) * g_ref[...] + b_ref[...]
    return jnp.where(z > 0, z, 0.2 * z)


def _knn_core(xtp, b, idx_ref, af_ref):
    n = xtp.shape[0]
    gram = lax.dot_general(xtp, xtp, (((1,), (1,)), ((), ())),
                           preferred_element_type=jnp.float32)
    rows = lax.broadcasted_iota(jnp.int32, (n, n), 0)
    cols = lax.broadcasted_iota(jnp.int32, (n, n), 1)
    colsf = cols.astype(jnp.float32)
    eyef = (rows == cols).astype(jnp.float32)
    xs = jnp.sum(xtp * xtp, axis=1, keepdims=True)
    xsr = lax.dot_general(xs, eyef, (((0,), (0,)), ((), ())),
                          preferred_element_type=jnp.float32,
                          precision=lax.Precision.HIGHEST)
    inner = -2.0 * gram
    pd = (-xs) - inner - xsr
    af_ref[...] = jnp.zeros((n, _KP), jnp.float32)
    nf = jnp.float32(n)
    for kk in range(_K):
        mrow = jnp.max(pd, axis=1, keepdims=True)
        jf = jnp.min(jnp.where(pd == mrow, colsf, nf), axis=1, keepdims=True)
        af_ref[:, kk:kk + 1] = jf
        pd = jnp.where(colsf == jf, -jnp.inf, pd)
    idxt = lax.dot_general(af_ref[...], eyef, (((0,), (0,)), ((), ())),
                           preferred_element_type=jnp.float32,
                           precision=lax.Precision.HIGHEST)
    idx_ref[0] = idxt.astype(jnp.int32) + b * n


def _a1_body(xtp_ref, idx_ref, af_ref):
    _knn_core(xtp_ref[0], pl.program_id(0), idx_ref, af_ref)


def _a_body(c, mx_ref, pstat_ref, g_ref, b_ref, xt_ref, idx_ref, af_ref):
    b = pl.program_id(0)
    m, v = _bn_scales(pstat_ref, g_ref, b_ref, _CNT)
    xtv = _normalize(mx_ref[0], m, v, g_ref, b_ref)
    xt_ref[0, :, 0:c] = xtv
    if c < 128:
        xt_ref[0, :, c:128] = jnp.zeros((_N, 128 - c), jnp.float32)
    _knn_core(xt_ref[0], b, idx_ref, af_ref)


def _run_a1(xtp):
    return pl.pallas_call(
        _a1_body,
        grid=(_B,),
        in_specs=[pl.BlockSpec((1, _N, 128), lambda b: (b, 0, 0))],
        out_specs=pl.BlockSpec((1, _KP, _N), lambda b: (b, 0, 0)),
        out_shape=jax.ShapeDtypeStruct((_B, _KP, _N), jnp.int32),
        scratch_shapes=[pltpu.VMEM((_N, _KP), jnp.float32)],
    )(xtp)


def _run_a(mx, pstat, g, b):
    c = mx.shape[2]
    return pl.pallas_call(
        functools.partial(_a_body, c),
        grid=(_B,),
        in_specs=[
            pl.BlockSpec((1, _N, c), lambda b: (b, 0, 0)),
            pl.BlockSpec((_B, 8, c), lambda b: (0, 0, 0)),
            pl.BlockSpec((1, c), lambda b: (0, 0)),
            pl.BlockSpec((1, c), lambda b: (0, 0)),
        ],
        out_specs=[
            pl.BlockSpec((1, _N, 128), lambda b: (b, 0, 0)),
            pl.BlockSpec((1, _KP, _N), lambda b: (b, 0, 0)),
        ],
        out_shape=[
            jax.ShapeDtypeStruct((_B, _N, 128), jnp.float32),
            jax.ShapeDtypeStruct((_B, _KP, _N), jnp.int32),
        ],
        scratch_shapes=[pltpu.VMEM((_N, _KP), jnp.float32)],
    )(mx, pstat, g, b)


def _sc_body(cw, xt_hbm, idx_hbm, diff_hbm, *scr):
    if cw == 128:
        idxb, rows, cb, sem0, sem1 = scr
        diffb = None
    else:
        idxb, rows, cb, diffb, sem0, sem1 = scr
    sems = (sem0, sem1)
    ch_sz = _CH if cw == 128 else 8
    seg = cw // 16
    ppw = (_B * _N) // _NW
    nch = ppw // ch_sz
    wid = lax.axis_index("s") * _NC + lax.axis_index("c")
    base = wid * ppw
    bb = base // _N
    nb = base - bb * _N
    pltpu.sync_copy(idx_hbm.at[bb, pl.ds(0, 24), pl.ds(nb, ppw)], idxb)

    def chunk_pair(i2, carry):
        c0 = i2 * 2
        hs = []
        for par in (0, 1):
            ch = c0 + par
            h_cb = pltpu.async_copy(
                xt_hbm.at[pl.ds(base + ch * ch_sz, ch_sz)], cb.at[par],
                sems[par])
            h_g = [pltpu.async_copy(
                xt_hbm.at[idxb.at[kk, pl.ds(ch * ch_sz, ch_sz)]],
                rows.at[par, kk], sems[par]) for kk in range(_K)]
            hs.append((h_cb, h_g))
        for par in (0, 1):
            ch = c0 + par
            h_cb, h_g = hs[par]
            h_cb.wait()
            for h in h_g:
                h.wait()

            def point(n, c2):
                for sg in range(seg):
                    sl = pl.ds(sg * 16, 16)
                    cv = cb[par, n, sl]
                    for kk in range(_K):
                        if cw == 128:
                            rows[par, kk, n, sl] = rows[par, kk, n, sl] - cv
                        else:
                            diffb[kk, n, sl] = rows[par, kk, n, sl] - cv
                return c2

            lax.fori_loop(0, ch_sz, point, 0)
            src = rows.at[par] if cw == 128 else diffb
            pltpu.sync_copy(
                src,
                diff_hbm.at[pl.ds(0, _K), pl.ds(base + ch * ch_sz, ch_sz)])
        return carry

    lax.fori_loop(0, nch // 2, chunk_pair, 0)


@functools.cache
def _make_sc_gather(cw):
    mesh = plsc.VectorSubcoreMesh(core_axis_name="c", subcore_axis_name="s")
    ch_sz = _CH if cw == 128 else 8
    scratch = [
        pltpu.VMEM((24, (_B * _N) // _NW), jnp.int32),
        pltpu.VMEM((2, _K, ch_sz, 128), jnp.float32),
        pltpu.VMEM((2, ch_sz, 128), jnp.float32),
    ]
    if cw != 128:
        scratch.append(pltpu.VMEM((_K, ch_sz, cw), jnp.float32))
    scratch += [pltpu.SemaphoreType.DMA, pltpu.SemaphoreType.DMA]
    return pl.kernel(
        functools.partial(_sc_body, cw),
        out_type=jax.ShapeDtypeStruct((_K, _B * _N, cw), jnp.float32),
        mesh=mesh,
        scratch_types=scratch,
    )


def _conv_body(diff_ref, xt_ref, wa_ref, wb_ref, mx_ref, pstat_ref):
    t = pl.program_id(1)
    yc = lax.dot_general(xt_ref[0], wb_ref[...], _DN,
                         preferred_element_type=jnp.float32)
    mx = None
    ss = None
    sq = None
    for kk in range(_K):
        yk = lax.dot_general(diff_ref[kk], wa_ref[...], _DN,
                             preferred_element_type=jnp.float32) + yc
        mx = yk if kk == 0 else jnp.maximum(mx, yk)
        ss = yk if kk == 0 else ss + yk
        sq = yk * yk if kk == 0 else sq + yk * yk
    mx_ref[0] = mx

    @pl.when(t == 0)
    def _init():
        pstat_ref[0] = jnp.zeros_like(pstat_ref[0])

    pstat_ref[0, 0:1, :] += jnp.sum(ss, axis=0, keepdims=True)
    pstat_ref[0, 1:2, :] += jnp.sum(sq, axis=0, keepdims=True)


def _run_conv(diff, xtp, wa, wb):
    cw, d = wa.shape
    return pl.pallas_call(
        _conv_body,
        grid=(_B, _NT),
        in_specs=[
            pl.BlockSpec((_K, _T, cw), lambda b, t: (0, b * _NT + t, 0)),
            pl.BlockSpec((1, _T, 128), lambda b, t: (b, t, 0)),
            pl.BlockSpec((cw, d), lambda b, t: (0, 0)),
            pl.BlockSpec((128, d), lambda b, t: (0, 0)),
        ],
        out_specs=[
            pl.BlockSpec((1, _T, d), lambda b, t: (b, t, 0)),
            pl.BlockSpec((1, 8, d), lambda b, t: (b, 0, 0)),
        ],
        out_shape=[
            jax.ShapeDtypeStruct((_B, _N, d), jnp.float32),
            jax.ShapeDtypeStruct((_B, 8, d), jnp.float32),
        ],
    )(diff, xtp, wa, wb)


def _final_body(x1_ref, x2_ref, x3_ref, m4_ref, pstat_ref, g4_ref, b4_ref,
                w5a_ref, w5b_ref, w5c_ref, w5d_ref, g5_ref, b5_ref, out_ref):
    m4, v4 = _bn_scales(pstat_ref, g4_ref, b4_ref, _CNT)
    ssum = None
    ssq = None
    mxs = []
    for b in range(_B):
        x4 = _normalize(m4_ref[b], m4, v4, g4_ref, b4_ref)
        y = (lax.dot_general(x1_ref[b][:, 0:64], w5a_ref[...], _DN,
                             preferred_element_type=jnp.float32)
             + lax.dot_general(x2_ref[b][:, 0:64], w5b_ref[...], _DN,
                               preferred_element_type=jnp.float32)
             + lax.dot_general(x3_ref[b][:, 0:128], w5c_ref[...], _DN,
                               preferred_element_type=jnp.float32)
             + lax.dot_general(x4, w5d_ref[...], _DN,
                               preferred_element_type=jnp.float32))
        s = jnp.sum(y, axis=0, keepdims=True)
        q = jnp.sum(y * y, axis=0, keepdims=True)
        ssum = s if b == 0 else ssum + s
        ssq = q if b == 0 else ssq + q
        mxs.append(jnp.max(y, axis=0, keepdims=True))
    mx = jnp.concatenate(mxs, axis=0)
    cnt = float(_B * _N)
    m5 = ssum / cnt
    v5 = ssq / cnt - m5 * m5
    out_ref[...] = _normalize(mx, m5, v5, g5_ref, b5_ref)


def _run_final(x1p, x2p, x3p, m4, pstat4, g4, b4, w5, g5, b5):
    return pl.pallas_call(
        _final_body,
        out_shape=jax.ShapeDtypeStruct((_B, 1024), jnp.float32),
    )(x1p, x2p, x3p, m4, pstat4, g4, b4,
      w5[0:64], w5[64:128], w5[128:256], w5[256:512], g5, b5)


def kernel(x, W1, g1, b1, W2, g2, b2, W3, g3, b3, W4, g4, b4, W5, g5, b5):
    f32 = jnp.float32
    row = lambda v: v.reshape(1, -1)
    xtp1 = jnp.zeros((_B, _N, 128), f32).at[:, :, :3].set(
        jnp.transpose(x, (0, 2, 1)))

    wa1 = jnp.zeros((16, 64), f32).at[:3].set(W1[:3])
    wb1 = jnp.zeros((128, 64), f32).at[:3].set(W1[3:])
    wa2 = W2[:64]
    wb2 = jnp.zeros((128, 64), f32).at[:64].set(W2[64:])
    wa3 = W3[:64]
    wb3 = jnp.zeros((128, 128), f32).at[:64].set(W3[64:])
    wa4 = W4[:128]
    wb4 = W4[128:]

    idx1 = _run_a1(xtp1)
    diff1 = _make_sc_gather(16)(xtp1.reshape(_B * _N, 128), idx1)
    m1, ps1 = _run_conv(diff1, xtp1, wa1, wb1)

    xtp2, idx2 = _run_a(m1, ps1, row(g1), row(b1))
    diff2 = _make_sc_gather(64)(xtp2.reshape(_B * _N, 128), idx2)
    m2, ps2 = _run_conv(diff2, xtp2, wa2, wb2)

    xtp3, idx3 = _run_a(m2, ps2, row(g2), row(b2))
    diff3 = _make_sc_gather(64)(xtp3.reshape(_B * _N, 128), idx3)
    m3, ps3 = _run_conv(diff3, xtp3, wa3, wb3)

    xtp4, idx4 = _run_a(m3, ps3, row(g3), row(b3))
    diff4 = _make_sc_gather(128)(xtp4.reshape(_B * _N, 128), idx4)
    m4, ps4 = _run_conv(diff4, xtp4, wa4, wb4)

    return _run_final(xtp2, xtp3, xtp4, m4, ps4, row(g4), row(b4),
                      W5, row(g5), row(b5))
